# Initial kernel scaffold; baseline (speedup 1.0000x reference)
import jax
import jax.numpy as jnp
from jax.experimental import pallas as pl


def kernel(x, params):
    raise NotImplementedError("write your pallas kernel here")


# speedup vs baseline: 3.0040x; 3.0040x over previous
import functools

import jax
import jax.numpy as jnp
import numpy as np
from jax.experimental import pallas as pl
from jax.experimental.pallas import tpu as pltpu

_PHI = (1.0 + 5.0 ** 0.5) / 2.0
_SQRT_PHI = float(np.sqrt(_PHI))
_D = 768
_E = 6
_KINDS = ('arith', 'general', 'geom', 'quantum', 'general', 'general')
_TBLK = 256
_NEG = -1e30


def _layernorm(v, g, b, eps=1e-5):
    m = jnp.mean(v, axis=-1, keepdims=True)
    c = v - m
    var = jnp.mean(c * c, axis=-1, keepdims=True)
    return c * jax.lax.rsqrt(var + eps) * g + b


def _gelu_exact(v):
    return 0.5 * v * (1.0 + jax.lax.erf(v * float(1.0 / np.sqrt(2.0))))


def _act1(kind, h):
    if kind == 'arith':
        return jax.nn.relu(h)
    if kind == 'quantum':
        return jnp.tanh(h)
    return _gelu_exact(h)


def _act2(kind, o):
    if kind == 'quantum':
        return jnp.tanh(o)
    if kind == 'geom':
        return _gelu_exact(o)
    return o


def _dot_t(a, b):
    return jax.lax.dot_general(a, b, (((1,), (1,)), ((), ())),
                               preferred_element_type=jnp.float32)


def _moe_kernel(x_ref, wg_ref, bg_ref, *refs):
    eargs = [refs[11 * e:11 * e + 11] for e in range(_E)]
    wcm, bcm, cg, cb = refs[11 * _E:11 * _E + 4]
    out_ref, conf_ref = refs[11 * _E + 4], refs[11 * _E + 5]

    x = x_ref[...]
    logits = _dot_t(x, wg_ref[...]) + bg_ref[...]
    col = jax.lax.broadcasted_iota(jnp.int32, logits.shape, 1)
    valid = col < _E
    lm = jnp.where(valid, logits, _NEG)
    m0 = jnp.max(lm, axis=1, keepdims=True)
    is0 = jnp.logical_and(lm == m0, valid)
    arg0 = jnp.min(jnp.where(is0, col, 127), axis=1, keepdims=True)
    lm1 = jnp.where(col == arg0, _NEG, lm)
    m1 = jnp.max(lm1, axis=1, keepdims=True)
    is1 = jnp.logical_and(lm1 == m1, jnp.logical_and(valid, col != arg0))
    arg1 = jnp.min(jnp.where(is1, col, 127), axis=1, keepdims=True)
    g0 = 1.0 / (1.0 + jnp.exp((m1 - m0) * _SQRT_PHI))
    g1 = 1.0 - g0

    combined = jnp.zeros_like(x)
    conf_acc = jnp.zeros((x.shape[0], 1), jnp.float32)
    for e in range(_E):
        w1, b1, w2, b2, wc1, bc1, wc2, bc2, spec, lng, lnb = eargs[e]
        we = g0 * (arg0 == e).astype(jnp.float32) + g1 * (arg1 == e).astype(jnp.float32)
        z = x + spec[...]
        h = _act1(_KINDS[e], _dot_t(z, w1[...]) + b1[...])
        o = _act2(_KINDS[e], _dot_t(h, w2[...]) + b2[...])
        o = _layernorm(o, lng[...], lnb[...])
        r = jax.nn.relu(_dot_t(o, wc1[...]) + bc1[...])
        clin = jnp.sum(r * wc2[...], axis=1, keepdims=True) + bc2[...]
        ce = jax.nn.sigmoid(clin)
        combined = combined + we * o
        conf_acc = conf_acc + we * ce

    y = _dot_t(combined, wcm[...]) + bcm[...]
    y = _layernorm(y, cg[...], cb[...])
    out_ref[...] = y
    conf_ref[...] = jnp.broadcast_to(conf_acc, (x.shape[0], 128))


def _full(shape):
    return pl.BlockSpec(shape, lambda i: (0,) * len(shape))


@jax.jit
def _moe_fused(x2d, wg, bg, flat_weights):
    n = x2d.shape[0]
    grid = (n // _TBLK,)
    in_specs = [pl.BlockSpec((_TBLK, _D), lambda i: (i, 0)),
                _full(wg.shape), _full(bg.shape)]
    in_specs += [_full(w.shape) for w in flat_weights]
    out_specs = [pl.BlockSpec((_TBLK, _D), lambda i: (i, 0)),
                 pl.BlockSpec((_TBLK, 128), lambda i: (i, 0))]
    out_shape = [jax.ShapeDtypeStruct((n, _D), jnp.float32),
                 jax.ShapeDtypeStruct((n, 128), jnp.float32)]
    out, conf = pl.pallas_call(
        _moe_kernel,
        grid=grid,
        in_specs=in_specs,
        out_specs=out_specs,
        out_shape=out_shape,
    )(x2d, wg, bg, *flat_weights)
    return out, conf[:, 0]


def kernel(x, params):
    b, s, d = x.shape
    x2d = x.reshape(b * s, d)
    gate = params['gate']
    wg = jnp.zeros((128, d), jnp.float32).at[:_E].set(gate['W'])
    bg = jnp.zeros((1, 128), jnp.float32).at[0, :_E].set(gate['b'])
    flat = []
    for e in range(_E):
        p = params['experts'][e]
        flat += [p['W1'], p['b1'][None, :], p['W2'], p['b2'][None, :],
                 p['Wc1'], p['bc1'][None, :], p['Wc2'], p['bc2'][None, :],
                 p['spec'][None, :], p['ln_g'][None, :], p['ln_b'][None, :]]
    cmb = params['combiner']
    flat += [cmb['W'], cmb['b'][None, :], cmb['ln_g'][None, :], cmb['ln_b'][None, :]]
    out, conf = _moe_fused(x2d, wg, bg, tuple(flat))
    return out.reshape(b, s, d), conf.reshape(b, s)
